# baseline (speedup 1.0000x reference)
import functools

import jax
import jax.numpy as jnp
from jax import lax
from jax.experimental import pallas as pl
from jax.experimental.pallas import tpu as pltpu
from jax.experimental.pallas import tpu_sc as plsc

N = 10000
E = 320000
D = 136
G = 64
HD = 68
HP = 80
NP = 10112

NC, NS = 2, 16
NW = NC * NS
CHUNK = 128
NB = 3
GRP = 32
NGRP = 5
CHE = GRP * NGRP
NCH = 80
EPAD = NS * CHE * CHUNK

RPT = NP // NS
BLK = 632
NBLK = NP // BLK


@functools.cache
def _get_deg_kernel():
    mesh = plsc.VectorSubcoreMesh(core_axis_name="c", subcore_axis_name="s")
    return functools.partial(
        pl.kernel,
        out_type=jax.ShapeDtypeStruct((NC, NP, 16), jnp.float32),
        mesh=mesh,
        compiler_params=pltpu.CompilerParams(use_tc_tiling_on_sc=False),
        scratch_types=[
            pltpu.VMEM((NCH, CHUNK), jnp.int32),
            pltpu.VMEM((CHUNK, 16), jnp.float32),
            pltpu.VMEM((8, 16), jnp.float32),
            pltpu.VMEM_SHARED((NP, 16), jnp.float32),
        ],
    )(_deg_body)


def _deg_body(dst_hbm, out_hbm, idx_v, ones_v, zbuf_v, acc_sh):
    c = lax.axis_index("c")
    s = lax.axis_index("s")
    wid = s * NC + c
    for r in range(8):
        zbuf_v[r, :] = jnp.zeros((16,), jnp.float32)
    for r in range(CHUNK):
        ones_v[r, :] = jnp.ones((16,), jnp.float32)
    for k in range(RPT // 8):
        pltpu.sync_copy(zbuf_v, acc_sh.at[pl.ds(s * RPT + k * 8, 8)])
    plsc.subcore_barrier()
    pltpu.sync_copy(dst_hbm.at[wid], idx_v)

    def body(j, _):
        pltpu.sync_copy(ones_v, acc_sh.at[idx_v.at[j]], add=True)
        return 0

    lax.fori_loop(0, NCH, body, 0)
    plsc.subcore_barrier()
    pltpu.sync_copy(acc_sh.at[pl.ds(s * RPT, RPT)],
                    out_hbm.at[c].at[pl.ds(s * RPT, RPT)])


@functools.cache
def _get_edge_kernel():
    mesh = plsc.VectorSubcoreMesh(core_axis_name="c", subcore_axis_name="s")
    return functools.partial(
        pl.kernel,
        out_type=jax.ShapeDtypeStruct((NC, NP, HP), jnp.float32),
        mesh=mesh,
        compiler_params=pltpu.CompilerParams(use_tc_tiling_on_sc=False),
        scratch_types=[
            pltpu.VMEM((CHE, CHUNK), jnp.int32),
            pltpu.VMEM((CHE, CHUNK), jnp.int32),
            pltpu.VMEM((CHUNK, HP), jnp.float32),
            pltpu.VMEM((CHUNK, HP), jnp.float32),
            pltpu.VMEM((CHUNK, HP), jnp.float32),
            pltpu.VMEM_SHARED((NP, HP), jnp.float32),
            pltpu.SemaphoreType.DMA,
            pltpu.SemaphoreType.DMA,
            pltpu.SemaphoreType.DMA,
            pltpu.SemaphoreType.DMA,
            pltpu.SemaphoreType.DMA,
            pltpu.SemaphoreType.DMA,
        ],
    )(_edge_body)


def _edge_body(y_hbm, src_hbm, dst_hbm, zero_hbm, out_hbm,
               src_v, dst_v, rows_0, rows_1, rows_2, acc_sh,
               gsem_0, gsem_1, gsem_2, ssem_0, ssem_1, ssem_2):
    c = lax.axis_index("c")
    s = lax.axis_index("s")
    wid = c * NS + s
    pltpu.sync_copy(zero_hbm, acc_sh.at[pl.ds(s * RPT, RPT)])
    pltpu.sync_copy(src_hbm.at[wid], src_v)
    pltpu.sync_copy(dst_hbm.at[s], dst_v)
    plsc.subcore_barrier()

    yc = y_hbm
    bufs = (rows_0, rows_1, rows_2)
    gsems = (gsem_0, gsem_1, gsem_2)
    ssems = (ssem_0, ssem_1, ssem_2)

    def group(g, _):
        base = g * GRP
        gdesc = [None] * NB
        sdesc = [None] * NB
        for k in range(GRP):
            b = k % NB
            if sdesc[b] is not None:
                sdesc[b].wait()
            gdesc[b] = pltpu.async_copy(
                yc.at[src_v.at[base + k]], bufs[b], gsems[b])
            j = k - (NB - 1)
            if j >= 0:
                bj = j % NB
                gdesc[bj].wait()
                sdesc[bj] = pltpu.async_copy(
                    bufs[bj], acc_sh.at[dst_v.at[base + j]], ssems[bj],
                    add=True)
        for j in range(GRP - NB + 1, GRP):
            bj = j % NB
            gdesc[bj].wait()
            sdesc[bj] = pltpu.async_copy(
                bufs[bj], acc_sh.at[dst_v.at[base + j]], ssems[bj], add=True)
        for d in sdesc:
            d.wait()
        return 0

    lax.fori_loop(0, NGRP, group, 0)
    plsc.subcore_barrier()
    pltpu.sync_copy(acc_sh.at[pl.ds(s * RPT, RPT)],
                    out_hbm.at[c].at[pl.ds(s * RPT, RPT)])


def _dinv_of(db):
    deg = db[0, :, 0:1] + db[1, :, 0:1] + 2.0
    return lax.rsqrt(deg)


def _prep_body(xb, wb, db, ob):
    dinv = _dinv_of(db[...])
    x = xb[...]
    w = wb[...]
    ob[0] = jnp.dot(x, w[0], preferred_element_type=jnp.float32) * dinv
    ob[1] = jnp.dot(x, w[1], preferred_element_type=jnp.float32) * dinv


def _halves(pb, yb, db, bb):
    dinv = _dinv_of(db[...])
    p = pb[...]
    y = yb[...]
    b = bb[...]
    h0 = jnp.maximum(dinv * (p[0] + 2.0 * y[0]) + b[0], 0.0)
    h1 = jnp.maximum(dinv * (p[1] + 2.0 * y[1]) + b[1], 0.0)
    return h0, h1, dinv


def _mid_body(pb, yb, db, wb, bb, ob):
    h0, h1, dinv = _halves(pb, yb, db, bb)
    w = wb[...]
    for c in range(2):
        xw = (jnp.dot(h0, w[0, c], preferred_element_type=jnp.float32)
              + jnp.dot(h1, w[1, c], preferred_element_type=jnp.float32))
        ob[c] = xw * dinv


def _final_body(pb, yb, db, bb, wb, batb, ob):
    i = pl.program_id(0)
    h0, h1, _ = _halves(pb, yb, db, bb)
    lane = lax.broadcasted_iota(jnp.int32, (1, HP), 1)
    haug0 = jnp.where(lane == HD, 1.0, h0)
    w = wb[...]
    z = (jnp.dot(haug0, w[0], preferred_element_type=jnp.float32)
         + jnp.dot(h1, w[1], preferred_element_type=jnp.float32))
    bat = batb[...][0]
    gid = lax.broadcasted_iota(jnp.int32, (G, BLK), 0)
    mask = (gid == bat).astype(jnp.float32)
    contrib = jnp.dot(mask, z, preferred_element_type=jnp.float32)

    @pl.when(i == 0)
    def _():
        ob[...] = jnp.zeros_like(ob)

    ob[...] += contrib

    @pl.when(i == NBLK - 1)
    def _():
        ob[...] = jax.nn.sigmoid(ob[...])


_SPEC_P = pl.BlockSpec((NC, BLK, HP), lambda i: (0, i, 0))
_SPEC_DEG = pl.BlockSpec((NC, BLK, 16), lambda i: (0, i, 0))
_SPEC_BIAS = pl.BlockSpec((NC, 1, HP), lambda i: (0, 0, 0))


def _full_spec(*dims):
    return pl.BlockSpec(dims, lambda i: (0,) * len(dims))


def _row_blocked(pallas_body, in_specs):
    return pl.pallas_call(
        pallas_body,
        grid=(NBLK,),
        in_specs=in_specs,
        out_specs=pl.BlockSpec((NC, BLK, HP), lambda i: (0, i, 0)),
        out_shape=jax.ShapeDtypeStruct((NC, NP, HP), jnp.float32),
    )


def _padh(a):
    if a.ndim == 1:
        return jnp.pad(a.reshape(2, 1, HD), ((0, 0), (0, 0), (0, HP - HD)))
    return jnp.pad(a.reshape(2, HD, -1), ((0, 0), (0, HP - HD), (0, 0)))


def kernel(x, edge_index, batch, W1, b1, W2, b2, Wl, bl):
    src = edge_index[0].astype(jnp.int32)
    dst = edge_index[1].astype(jnp.int32)
    pad = EPAD - E
    dump = N + (jnp.arange(pad, dtype=jnp.int32) % (NP - N))
    srcp = jnp.concatenate([src, dump])
    dstp = jnp.concatenate([dst, dump])
    srcw = jnp.stack([srcp, srcp + NP]).reshape(NC * NS, CHE, CHUNK)
    dstw = dstp.reshape(NS, CHE, CHUNK)
    dstd = dstp.reshape(NW, NCH, CHUNK)

    xpad = jnp.pad(x, ((0, NP - N), (0, 0)))
    W1h = jnp.pad(W1.reshape(D, 2, HD).transpose(1, 0, 2),
                  ((0, 0), (0, 0), (0, HP - HD)))
    W2q = jnp.pad(W2.reshape(2, HD, 2, HD).transpose(0, 2, 1, 3),
                  ((0, 0), (0, 0), (0, HP - HD), (0, HP - HD)))
    b1h = _padh(b1)
    b2h = _padh(b2)
    Wlq = _padh(Wl.reshape(-1))
    Wlq = Wlq.at[0, 0, HD].set(bl[0]).transpose(0, 2, 1)
    batp = jnp.pad(batch.astype(jnp.int32), (0, NP - N),
                   constant_values=G).reshape(NBLK, 1, BLK)
    zrows = jnp.zeros((RPT, HP), jnp.float32)

    deg = _get_deg_kernel()(dstd)

    y1 = _row_blocked(
        _prep_body,
        [pl.BlockSpec((BLK, D), lambda i: (i, 0)),
         _full_spec(NC, D, HP),
         _SPEC_DEG],
    )(xpad, W1h, deg)

    p1 = _get_edge_kernel()(y1.reshape(NC * NP, HP), srcw, dstw, zrows)

    y2 = _row_blocked(
        _mid_body,
        [_SPEC_P, _SPEC_P, _SPEC_DEG, _full_spec(NC, NC, HP, HP),
         _SPEC_BIAS],
    )(p1, y1, deg, W2q, b1h)

    p2 = _get_edge_kernel()(y2.reshape(NC * NP, HP), srcw, dstw, zrows)

    pooled = pl.pallas_call(
        _final_body,
        grid=(NBLK,),
        in_specs=[_SPEC_P, _SPEC_P, _SPEC_DEG, _SPEC_BIAS,
                  _full_spec(NC, HP, 1),
                  pl.BlockSpec((1, 1, BLK), lambda i: (i, 0, 0))],
        out_specs=pl.BlockSpec((G, 1), lambda i: (0, 0)),
        out_shape=jax.ShapeDtypeStruct((G, 1), jnp.float32),
    )(p2, y2, deg, b2h, Wlq, batp)

    return pooled.reshape(-1)
